# baseline (speedup 1.0000x reference)
import functools

import jax
import jax.numpy as jnp
from jax import lax
from jax.experimental import pallas as pl
from jax.experimental.pallas import tpu as pltpu
from jax.experimental.pallas import tpu_sc as plsc

_NUM_CF = 8
_NUM_DF = 26
_VOCAB = 100000
_EMBED = 64
_BATCH = 16384

_INFO = plsc.get_sparse_core_info()
_NC = _INFO.num_cores
_NS = _INFO.num_subcores
_NW = _NC * _NS
_BPW = _BATCH // _NW
_IDXW = 128
_STG = 64
_NSTG = (_NUM_DF * _BPW) // _STG

_SPLIT = 49920
_PROWS = _VOCAB - _SPLIT
_ZPAD = 8
_PROWSP = _PROWS + _ZPAD
_DUMMY = _PROWS
_CHUNK = 2048
_NFULL = _PROWS // _CHUNK
_TAIL = _PROWS - _NFULL * _CHUNK


def _xpose_body(in_ref, out_hbm, ybuf0, ybuf1, sem0, sem1):
    i = pl.program_id(0)
    ybufs = (ybuf0, ybuf1)
    sems = (sem0, sem1)
    x = in_ref

    prev = [None, None]
    for k in range(_NFULL + 2):
        b = k % 2
        a = k * _CHUNK
        if prev[b] is not None:
            prev[b].wait()
        if k <= _NFULL:
            n = _CHUNK if k < _NFULL else _TAIL
            xlo = x[0, :, a:a + n]
            xhi = x[0, :, _SPLIT + a:_SPLIT + a + n]
            y = jnp.transpose(
                jnp.concatenate([xlo, xhi], axis=0), (1, 0))
            ybufs[b][0:n, :] = y
            dst = out_hbm.at[i, pl.ds(a, n)]
        else:
            n = _ZPAD
            ybufs[b][0:n, :] = jnp.zeros((_ZPAD, 2 * _EMBED), jnp.float32)
            dst = out_hbm.at[i, pl.ds(_PROWS, n)]
        cp = pltpu.make_async_copy(ybufs[b].at[pl.ds(0, n)], dst, sems[b])
        cp.start()
        prev[b] = cp
    for b in range(2):
        if prev[b] is not None:
            prev[b].wait()


def _tc_format_table(tables_t):
    return pl.pallas_call(
        _xpose_body,
        grid=(_NUM_DF,),
        in_specs=[pl.BlockSpec((1, _EMBED, _VOCAB), lambda i: (i, 0, 0))],
        out_specs=pl.BlockSpec(memory_space=pl.ANY),
        out_shape=jax.ShapeDtypeStruct((_NUM_DF, _PROWSP, 2 * _EMBED),
                                       jnp.float32),
        scratch_shapes=[
            pltpu.VMEM((_CHUNK, 2 * _EMBED), jnp.float32),
            pltpu.VMEM((_CHUNK, 2 * _EMBED), jnp.float32),
            pltpu.SemaphoreType.DMA,
            pltpu.SemaphoreType.DMA,
        ],
        compiler_params=pltpu.CompilerParams(
            vmem_limit_bytes=60 * 1024 * 1024),
    )(tables_t)


def _sc_embsum(didxw, tables_p):
    mesh = plsc.VectorSubcoreMesh(core_axis_name="c", subcore_axis_name="s")

    @functools.partial(
        pl.kernel,
        out_type=jax.ShapeDtypeStruct((_BATCH, _EMBED), jnp.float32),
        mesh=mesh,
        scratch_types=[
            pltpu.VMEM((_NUM_DF * _BPW,), jnp.int32),
            pltpu.VMEM((_STG,), jnp.int32),
            pltpu.VMEM((_STG,), jnp.int32),
            pltpu.VMEM((_STG,), jnp.int32),
            pltpu.VMEM((_STG,), jnp.int32),
            pltpu.VMEM((_STG, 2 * _EMBED), jnp.float32),
            pltpu.VMEM((_STG, 2 * _EMBED), jnp.float32),
            pltpu.VMEM((_STG, 2 * _EMBED), jnp.float32),
            pltpu.VMEM((_STG, 2 * _EMBED), jnp.float32),
            pltpu.VMEM((_BPW, _EMBED), jnp.float32),
            pltpu.SemaphoreType.DMA,
            pltpu.SemaphoreType.DMA,
        ],
    )
    def body(didx_hbm, tab_hbm, out_hbm,
             rawbig, lo0, lo1, hi0, hi1, rlo0, rlo1, rhi0, rhi1, acc,
             sem0, sem1):
        wid = lax.axis_index("s") * _NC + lax.axis_index("c")
        lo_bufs = (lo0, lo1)
        hi_bufs = (hi0, hi1)
        rlo_bufs = (rlo0, rlo1)
        rhi_bufs = (rhi0, rhi1)
        sems = (sem0, sem1)
        per_feat = _BPW // _STG

        pltpu.sync_copy(didx_hbm.at[wid], rawbig)

        zero16 = jnp.zeros((16,), jnp.float32)

        def stage_and_fire(h, slot):
            i = h // per_feat
            lob, hib = lo_bufs[slot], hi_bufs[slot]
            for c in range(_STG // 16):
                v = rawbig[pl.ds(h * _STG + c * 16, 16)]
                is_lo = v < _SPLIT
                lob[pl.ds(c * 16, 16)] = jnp.where(is_lo, v, _DUMMY)
                hib[pl.ds(c * 16, 16)] = jnp.where(is_lo, _DUMMY,
                                                   v - _SPLIT)
            pltpu.make_async_copy(
                tab_hbm.at[i].at[lob], rlo_bufs[slot], sems[slot]).start()
            pltpu.make_async_copy(
                tab_hbm.at[i].at[hib], rhi_bufs[slot], sems[slot]).start()

        def wait_gathers(h, slot):
            i = h // per_feat
            pltpu.make_async_copy(
                tab_hbm.at[i].at[lo_bufs[slot]], rlo_bufs[slot],
                sems[slot]).wait()
            pltpu.make_async_copy(
                tab_hbm.at[i].at[hi_bufs[slot]], rhi_bufs[slot],
                sems[slot]).wait()

        def accumulate(h, slot):
            rlo, rhi = rlo_bufs[slot], rhi_bufs[slot]
            sub = h % per_feat

            def accrow(r, _):
                arow = sub * _STG + r
                for c in range(_EMBED // 16):
                    plsc.addupdate(
                        acc.at[arow, pl.ds(c * 16, 16)],
                        rlo[r, pl.ds(c * 16, 16)]
                        + rhi[r, pl.ds(_EMBED + c * 16, 16)])
                return 0

            lax.fori_loop(0, _STG, accrow, 0, unroll=4)

        stage_and_fire(0, 0)

        def zrow(r, _):
            for c in range(_EMBED // 16):
                acc[r, pl.ds(c * 16, 16)] = zero16
            return 0

        lax.fori_loop(0, _BPW, zrow, 0, unroll=4)

        def loop_body(h, _):
            for slot in range(2):

                @pl.when(h % 2 == slot)
                def _(slot=slot):
                    wait_gathers(h, slot)
                    @pl.when(h + 1 < _NSTG)
                    def _():
                        stage_and_fire(h + 1, 1 - slot)
                    accumulate(h, slot)

            return 0

        lax.fori_loop(0, _NSTG, loop_body, 0)
        pltpu.sync_copy(acc, out_hbm.at[pl.ds(wid * _BPW, _BPW)])

    return body(didxw, tables_p)


def _tc_mlp_body(cf_ref, w1_ref, b1_ref, w2_ref, b2_ref, emb_ref, out_ref):
    x = cf_ref[...]
    x = jnp.where(jnp.isnan(x), 0.0, x)
    h = jnp.maximum(
        jnp.dot(x, w1_ref[...], preferred_element_type=jnp.float32)
        + b1_ref[...], 0.0)
    h = jnp.clip(h, -65000.0, 65000.0)
    o = jnp.maximum(
        jnp.dot(h, w2_ref[...], preferred_element_type=jnp.float32)
        + b2_ref[...], 0.0)
    out_ref[...] = o + emb_ref[...]


def _tc_mlp(cf_mat, w1t, b1, w2t, b2, embsum):
    blk = 2048
    grid = _BATCH // blk
    return pl.pallas_call(
        _tc_mlp_body,
        grid=(grid,),
        in_specs=[
            pl.BlockSpec((blk, _NUM_CF), lambda i: (i, 0)),
            pl.BlockSpec((_NUM_CF, 2 * _NUM_CF), lambda i: (0, 0)),
            pl.BlockSpec((1, 2 * _NUM_CF), lambda i: (0, 0)),
            pl.BlockSpec((2 * _NUM_CF, _EMBED), lambda i: (0, 0)),
            pl.BlockSpec((1, _EMBED), lambda i: (0, 0)),
            pl.BlockSpec((blk, _EMBED), lambda i: (i, 0)),
        ],
        out_specs=pl.BlockSpec((blk, _EMBED), lambda i: (i, 0)),
        out_shape=jax.ShapeDtypeStruct((_BATCH, _EMBED), jnp.float32),
    )(cf_mat, w1t, b1.reshape(1, -1), w2t, b2.reshape(1, -1), embsum)


def kernel(cf_0, cf_1, cf_2, cf_3, cf_4, cf_5, cf_6, cf_7,
           df_0, df_1, df_2, df_3, df_4, df_5, df_6, df_7, df_8, df_9,
           df_10, df_11, df_12, df_13, df_14, df_15, df_16, df_17, df_18,
           df_19, df_20, df_21, df_22, df_23, df_24, df_25,
           W1, b1, W2, b2, tables):
    cfs = [cf_0, cf_1, cf_2, cf_3, cf_4, cf_5, cf_6, cf_7]
    dfs = [df_0, df_1, df_2, df_3, df_4, df_5, df_6, df_7, df_8, df_9,
           df_10, df_11, df_12, df_13, df_14, df_15, df_16, df_17, df_18,
           df_19, df_20, df_21, df_22, df_23, df_24, df_25]
    cf_mat = jnp.stack(cfs, axis=1)
    didxw = (jnp.stack(dfs, axis=0)
             .reshape(_NUM_DF, _NW, _BPW)
             .transpose(1, 0, 2)
             .reshape(_NW, _NUM_DF * _BPW))
    tables_t = jnp.transpose(tables, (0, 2, 1))
    tables_p = _tc_format_table(tables_t)
    embsum = _sc_embsum(didxw, tables_p)
    return _tc_mlp(cf_mat, W1.T, b1, W2.T, b2, embsum)


# speedup vs baseline: 6.3267x; 6.3267x over previous
import functools

import jax
import jax.numpy as jnp
from jax import lax
from jax.experimental import pallas as pl
from jax.experimental.pallas import tpu as pltpu
from jax.experimental.pallas import tpu_sc as plsc

_NUM_CF = 8
_NUM_DF = 26
_VOCAB = 100000
_EMBED = 64
_BATCH = 16384

_INFO = plsc.get_sparse_core_info()
_NC = _INFO.num_cores
_NS = _INFO.num_subcores
_NW = _NC * _NS
_BPW = _BATCH // _NW
_IDXW = 128
_STG = 64
_NSTG = (_NUM_DF * _BPW) // _STG

_SPLIT = 49920
_PROWS = _VOCAB - _SPLIT
_ZPAD = 2048
_PROWSP = _PROWS + _ZPAD
_DUMMY = _PROWS
_CHUNK = 2048
_NFULL = _PROWS // _CHUNK
_TAIL = _PROWS - _NFULL * _CHUNK


def _xpose_body(in_ref, out_hbm, ybuf0, ybuf1, sem0, sem1):
    i = pl.program_id(0)
    ybufs = (ybuf0, ybuf1)
    sems = (sem0, sem1)
    x = in_ref

    prev = [None, None]
    for k in range(_NFULL + 2):
        b = k % 2
        a = k * _CHUNK
        if prev[b] is not None:
            prev[b].wait()
        if k <= _NFULL:
            n = _CHUNK if k < _NFULL else _TAIL
            xlo = x[0, :, a:a + n]
            xhi = x[0, :, _SPLIT + a:_SPLIT + a + n]
            y = jnp.transpose(
                jnp.concatenate([xlo, xhi], axis=0), (1, 0))
            ybufs[b][0:n, :] = y
            dst = out_hbm.at[i, pl.ds(a, n)]
        else:
            n = _ZPAD
            ybufs[b][0:n, :] = jnp.zeros((_ZPAD, 2 * _EMBED), jnp.float32)
            dst = out_hbm.at[i, pl.ds(_PROWS, n)]
            assert _ZPAD <= _CHUNK
        cp = pltpu.make_async_copy(ybufs[b].at[pl.ds(0, n)], dst, sems[b])
        cp.start()
        prev[b] = cp
    for b in range(2):
        if prev[b] is not None:
            prev[b].wait()


def _tc_format_table(tables_t):
    return pl.pallas_call(
        _xpose_body,
        grid=(_NUM_DF,),
        in_specs=[pl.BlockSpec((1, _EMBED, _VOCAB), lambda i: (i, 0, 0))],
        out_specs=pl.BlockSpec(memory_space=pl.ANY),
        out_shape=jax.ShapeDtypeStruct((_NUM_DF, _PROWSP, 2 * _EMBED),
                                       jnp.float32),
        scratch_shapes=[
            pltpu.VMEM((_CHUNK, 2 * _EMBED), jnp.float32),
            pltpu.VMEM((_CHUNK, 2 * _EMBED), jnp.float32),
            pltpu.SemaphoreType.DMA,
            pltpu.SemaphoreType.DMA,
        ],
        compiler_params=pltpu.CompilerParams(
            vmem_limit_bytes=60 * 1024 * 1024),
    )(tables_t)


def _sc_embsum(didxw, tables_p):
    mesh = plsc.VectorSubcoreMesh(core_axis_name="c", subcore_axis_name="s")

    @functools.partial(
        pl.kernel,
        out_type=jax.ShapeDtypeStruct((_BATCH, _EMBED), jnp.float32),
        mesh=mesh,
        scratch_types=[
            pltpu.VMEM((_NUM_DF * _BPW,), jnp.int32),
            pltpu.VMEM((_STG,), jnp.int32),
            pltpu.VMEM((_STG,), jnp.int32),
            pltpu.VMEM((_STG,), jnp.int32),
            pltpu.VMEM((_STG,), jnp.int32),
            pltpu.VMEM((_STG, 2 * _EMBED), jnp.float32),
            pltpu.VMEM((_STG, 2 * _EMBED), jnp.float32),
            pltpu.VMEM((_STG, 2 * _EMBED), jnp.float32),
            pltpu.VMEM((_STG, 2 * _EMBED), jnp.float32),
            pltpu.VMEM((_BPW, _EMBED), jnp.float32),
            pltpu.SemaphoreType.DMA,
            pltpu.SemaphoreType.DMA,
        ],
    )
    def body(didx_hbm, tab_hbm, out_hbm,
             rawbig, lo0, lo1, hi0, hi1, rlo0, rlo1, rhi0, rhi1, acc,
             sem0, sem1):
        wid = lax.axis_index("s") * _NC + lax.axis_index("c")
        lo_bufs = (lo0, lo1)
        hi_bufs = (hi0, hi1)
        rlo_bufs = (rlo0, rlo1)
        rhi_bufs = (rhi0, rhi1)
        sems = (sem0, sem1)
        per_feat = _BPW // _STG

        pltpu.sync_copy(didx_hbm.at[wid], rawbig)

        zero16 = jnp.zeros((16,), jnp.float32)

        def stage_and_fire(h, slot):
            i = h // per_feat
            lob, hib = lo_bufs[slot], hi_bufs[slot]
            for c in range(_STG // 16):
                v = rawbig[pl.ds(h * _STG + c * 16, 16)]
                is_lo = v < _SPLIT
                dm = _DUMMY + (v & (_ZPAD - 1))
                lob[pl.ds(c * 16, 16)] = jnp.where(is_lo, v, dm)
                hib[pl.ds(c * 16, 16)] = jnp.where(is_lo, dm,
                                                   v - _SPLIT)
            pltpu.make_async_copy(
                tab_hbm.at[i].at[lob], rlo_bufs[slot], sems[slot]).start()
            pltpu.make_async_copy(
                tab_hbm.at[i].at[hib], rhi_bufs[slot], sems[slot]).start()

        def wait_gathers(h, slot):
            i = h // per_feat
            pltpu.make_async_copy(
                tab_hbm.at[i].at[lo_bufs[slot]], rlo_bufs[slot],
                sems[slot]).wait()
            pltpu.make_async_copy(
                tab_hbm.at[i].at[hi_bufs[slot]], rhi_bufs[slot],
                sems[slot]).wait()

        def accumulate(h, slot):
            rlo, rhi = rlo_bufs[slot], rhi_bufs[slot]
            sub = h % per_feat

            def accrow(r, _):
                arow = sub * _STG + r
                for c in range(_EMBED // 16):
                    plsc.addupdate(
                        acc.at[arow, pl.ds(c * 16, 16)],
                        rlo[r, pl.ds(c * 16, 16)]
                        + rhi[r, pl.ds(_EMBED + c * 16, 16)])
                return 0

            lax.fori_loop(0, _STG, accrow, 0, unroll=4)

        stage_and_fire(0, 0)

        def zrow(r, _):
            for c in range(_EMBED // 16):
                acc[r, pl.ds(c * 16, 16)] = zero16
            return 0

        lax.fori_loop(0, _BPW, zrow, 0, unroll=4)

        def loop_body(h, _):
            for slot in range(2):

                @pl.when(h % 2 == slot)
                def _(slot=slot):
                    wait_gathers(h, slot)
                    @pl.when(h + 1 < _NSTG)
                    def _():
                        stage_and_fire(h + 1, 1 - slot)
                    accumulate(h, slot)

            return 0

        lax.fori_loop(0, _NSTG, loop_body, 0)
        pltpu.sync_copy(acc, out_hbm.at[pl.ds(wid * _BPW, _BPW)])

    return body(didxw, tables_p)


def _tc_mlp_body(cf_ref, w1_ref, b1_ref, w2_ref, b2_ref, emb_ref, out_ref):
    x = cf_ref[...]
    x = jnp.where(jnp.isnan(x), 0.0, x)
    h = jnp.maximum(
        jnp.dot(x, w1_ref[...], preferred_element_type=jnp.float32)
        + b1_ref[...], 0.0)
    h = jnp.clip(h, -65000.0, 65000.0)
    o = jnp.maximum(
        jnp.dot(h, w2_ref[...], preferred_element_type=jnp.float32)
        + b2_ref[...], 0.0)
    out_ref[...] = o + emb_ref[...]


def _tc_mlp(cf_mat, w1t, b1, w2t, b2, embsum):
    blk = 2048
    grid = _BATCH // blk
    return pl.pallas_call(
        _tc_mlp_body,
        grid=(grid,),
        in_specs=[
            pl.BlockSpec((blk, _NUM_CF), lambda i: (i, 0)),
            pl.BlockSpec((_NUM_CF, 2 * _NUM_CF), lambda i: (0, 0)),
            pl.BlockSpec((1, 2 * _NUM_CF), lambda i: (0, 0)),
            pl.BlockSpec((2 * _NUM_CF, _EMBED), lambda i: (0, 0)),
            pl.BlockSpec((1, _EMBED), lambda i: (0, 0)),
            pl.BlockSpec((blk, _EMBED), lambda i: (i, 0)),
        ],
        out_specs=pl.BlockSpec((blk, _EMBED), lambda i: (i, 0)),
        out_shape=jax.ShapeDtypeStruct((_BATCH, _EMBED), jnp.float32),
    )(cf_mat, w1t, b1.reshape(1, -1), w2t, b2.reshape(1, -1), embsum)


def kernel(cf_0, cf_1, cf_2, cf_3, cf_4, cf_5, cf_6, cf_7,
           df_0, df_1, df_2, df_3, df_4, df_5, df_6, df_7, df_8, df_9,
           df_10, df_11, df_12, df_13, df_14, df_15, df_16, df_17, df_18,
           df_19, df_20, df_21, df_22, df_23, df_24, df_25,
           W1, b1, W2, b2, tables):
    cfs = [cf_0, cf_1, cf_2, cf_3, cf_4, cf_5, cf_6, cf_7]
    dfs = [df_0, df_1, df_2, df_3, df_4, df_5, df_6, df_7, df_8, df_9,
           df_10, df_11, df_12, df_13, df_14, df_15, df_16, df_17, df_18,
           df_19, df_20, df_21, df_22, df_23, df_24, df_25]
    cf_mat = jnp.stack(cfs, axis=1)
    didxw = (jnp.stack(dfs, axis=0)
             .reshape(_NUM_DF, _NW, _BPW)
             .transpose(1, 0, 2)
             .reshape(_NW, _NUM_DF * _BPW))
    tables_t = jnp.transpose(tables, (0, 2, 1))
    tables_p = _tc_format_table(tables_t)
    embsum = _sc_embsum(didxw, tables_p)
    return _tc_mlp(cf_mat, W1.T, b1, W2.T, b2, embsum)


# speedup vs baseline: 6.8730x; 1.0863x over previous
import functools

import jax
import jax.numpy as jnp
from jax import lax
from jax.experimental import pallas as pl
from jax.experimental.pallas import tpu as pltpu
from jax.experimental.pallas import tpu_sc as plsc

_NUM_CF = 8
_NUM_DF = 26
_VOCAB = 100000
_EMBED = 64
_BATCH = 16384

_INFO = plsc.get_sparse_core_info()
_NC = _INFO.num_cores
_NS = _INFO.num_subcores
_NW = _NC * _NS
_BPW = _BATCH // _NW
_IDXW = 128
_STG = 64
_GRP = 2
_NGRP = _NUM_DF // _GRP
_NSTG = (_GRP * _BPW) // _STG

_SPLIT = 49920
_PROWS = _VOCAB - _SPLIT
_ZPAD = 2048
_PROWSP = _PROWS + _ZPAD
_DUMMY = _PROWS
_CHUNK = 2048
_NFULL = _PROWS // _CHUNK
_TAIL = _PROWS - _NFULL * _CHUNK


def _xpose_body(in_ref, out_hbm, ybuf0, ybuf1, sem0, sem1):
    i = pl.program_id(0)
    ybufs = (ybuf0, ybuf1)
    sems = (sem0, sem1)
    x = in_ref

    prev = [None, None]
    for k in range(_NFULL + 2):
        b = k % 2
        a = k * _CHUNK
        if prev[b] is not None:
            prev[b].wait()
        if k <= _NFULL:
            n = _CHUNK if k < _NFULL else _TAIL
            xlo = x[0, :, a:a + n]
            xhi = x[0, :, _SPLIT + a:_SPLIT + a + n]
            y = jnp.transpose(
                jnp.concatenate([xlo, xhi], axis=0), (1, 0))
            ybufs[b][0:n, :] = y
            dst = out_hbm.at[i, pl.ds(a, n)]
        else:
            n = _ZPAD
            ybufs[b][0:n, :] = jnp.zeros((_ZPAD, 2 * _EMBED), jnp.float32)
            dst = out_hbm.at[i, pl.ds(_PROWS, n)]
            assert _ZPAD <= _CHUNK
        cp = pltpu.make_async_copy(ybufs[b].at[pl.ds(0, n)], dst, sems[b])
        cp.start()
        prev[b] = cp
    for b in range(2):
        if prev[b] is not None:
            prev[b].wait()


def _tc_format_table(tables_t, g):
    return pl.pallas_call(
        _xpose_body,
        grid=(_GRP,),
        in_specs=[pl.BlockSpec((1, _EMBED, _VOCAB),
                               lambda i, g=g: (g * _GRP + i, 0, 0))],
        out_specs=pl.BlockSpec(memory_space=pl.ANY),
        out_shape=jax.ShapeDtypeStruct((_GRP, _PROWSP, 2 * _EMBED),
                                       jnp.float32),
        scratch_shapes=[
            pltpu.VMEM((_CHUNK, 2 * _EMBED), jnp.float32),
            pltpu.VMEM((_CHUNK, 2 * _EMBED), jnp.float32),
            pltpu.SemaphoreType.DMA,
            pltpu.SemaphoreType.DMA,
        ],
        compiler_params=pltpu.CompilerParams(
            vmem_limit_bytes=60 * 1024 * 1024),
    )(tables_t)


def _sc_embsum(didxf, tables_p, g):
    mesh = plsc.VectorSubcoreMesh(core_axis_name="c", subcore_axis_name="s")

    @functools.partial(
        pl.kernel,
        out_type=jax.ShapeDtypeStruct((_BATCH, _EMBED), jnp.float32),
        mesh=mesh,
        scratch_types=[
            pltpu.VMEM((_GRP * _BPW,), jnp.int32),
            pltpu.VMEM((_STG,), jnp.int32),
            pltpu.VMEM((_STG,), jnp.int32),
            pltpu.VMEM((_STG,), jnp.int32),
            pltpu.VMEM((_STG,), jnp.int32),
            pltpu.VMEM((_STG, 2 * _EMBED), jnp.float32),
            pltpu.VMEM((_STG, 2 * _EMBED), jnp.float32),
            pltpu.VMEM((_STG, 2 * _EMBED), jnp.float32),
            pltpu.VMEM((_STG, 2 * _EMBED), jnp.float32),
            pltpu.VMEM((_BPW, _EMBED), jnp.float32),
            pltpu.SemaphoreType.DMA,
            pltpu.SemaphoreType.DMA,
        ],
    )
    def body(didx_hbm, tab_hbm, out_hbm,
             rawbig, lo0, lo1, hi0, hi1, rlo0, rlo1, rhi0, rhi1, acc,
             sem0, sem1):
        wid = lax.axis_index("s") * _NC + lax.axis_index("c")
        lo_bufs = (lo0, lo1)
        hi_bufs = (hi0, hi1)
        rlo_bufs = (rlo0, rlo1)
        rhi_bufs = (rhi0, rhi1)
        sems = (sem0, sem1)
        per_feat = _BPW // _STG

        pltpu.sync_copy(
            didx_hbm.at[pl.ds(wid * (_NUM_DF * _BPW) + g * (_GRP * _BPW),
                              _GRP * _BPW)],
            rawbig)

        zero16 = jnp.zeros((16,), jnp.float32)

        def stage_and_fire(h, slot):
            i = h // per_feat
            lob, hib = lo_bufs[slot], hi_bufs[slot]
            for c in range(_STG // 16):
                v = rawbig[pl.ds(h * _STG + c * 16, 16)]
                is_lo = v < _SPLIT
                dm = _DUMMY + (v & (_ZPAD - 1))
                lob[pl.ds(c * 16, 16)] = jnp.where(is_lo, v, dm)
                hib[pl.ds(c * 16, 16)] = jnp.where(is_lo, dm,
                                                   v - _SPLIT)
            pltpu.make_async_copy(
                tab_hbm.at[i].at[lob], rlo_bufs[slot], sems[slot]).start()
            pltpu.make_async_copy(
                tab_hbm.at[i].at[hib], rhi_bufs[slot], sems[slot]).start()

        def wait_gathers(h, slot):
            i = h // per_feat
            pltpu.make_async_copy(
                tab_hbm.at[i].at[lo_bufs[slot]], rlo_bufs[slot],
                sems[slot]).wait()
            pltpu.make_async_copy(
                tab_hbm.at[i].at[hi_bufs[slot]], rhi_bufs[slot],
                sems[slot]).wait()

        def accumulate(h, slot):
            rlo, rhi = rlo_bufs[slot], rhi_bufs[slot]
            sub = h % per_feat

            def accrow(r, _):
                arow = sub * _STG + r
                for c in range(_EMBED // 16):
                    plsc.addupdate(
                        acc.at[arow, pl.ds(c * 16, 16)],
                        rlo[r, pl.ds(c * 16, 16)]
                        + rhi[r, pl.ds(_EMBED + c * 16, 16)])
                return 0

            lax.fori_loop(0, _STG, accrow, 0, unroll=4)

        stage_and_fire(0, 0)

        def zrow(r, _):
            for c in range(_EMBED // 16):
                acc[r, pl.ds(c * 16, 16)] = zero16
            return 0

        lax.fori_loop(0, _BPW, zrow, 0, unroll=4)

        def loop_body(h, _):
            for slot in range(2):

                @pl.when(h % 2 == slot)
                def _(slot=slot):
                    wait_gathers(h, slot)
                    @pl.when(h + 1 < _NSTG)
                    def _():
                        stage_and_fire(h + 1, 1 - slot)
                    accumulate(h, slot)

            return 0

        lax.fori_loop(0, _NSTG, loop_body, 0)
        pltpu.sync_copy(acc, out_hbm.at[pl.ds(wid * _BPW, _BPW)])

    return body(didxf, tables_p)


def _tc_mlp_body(cf_ref, w1_ref, b1_ref, w2_ref, b2_ref, *rest):
    emb_refs, out_ref = rest[:-1], rest[-1]
    x = cf_ref[...]
    x = jnp.where(jnp.isnan(x), 0.0, x)
    h = jnp.maximum(
        jnp.dot(x, w1_ref[...], preferred_element_type=jnp.float32)
        + b1_ref[...], 0.0)
    h = jnp.clip(h, -65000.0, 65000.0)
    o = jnp.maximum(
        jnp.dot(h, w2_ref[...], preferred_element_type=jnp.float32)
        + b2_ref[...], 0.0)
    for e in emb_refs:
        o = o + e[...]
    out_ref[...] = o


def _tc_mlp(cf_mat, w1t, b1, w2t, b2, embsums):
    blk = 2048
    grid = _BATCH // blk
    return pl.pallas_call(
        _tc_mlp_body,
        grid=(grid,),
        in_specs=[
            pl.BlockSpec((blk, _NUM_CF), lambda i: (i, 0)),
            pl.BlockSpec((_NUM_CF, 2 * _NUM_CF), lambda i: (0, 0)),
            pl.BlockSpec((1, 2 * _NUM_CF), lambda i: (0, 0)),
            pl.BlockSpec((2 * _NUM_CF, _EMBED), lambda i: (0, 0)),
            pl.BlockSpec((1, _EMBED), lambda i: (0, 0)),
        ] + [pl.BlockSpec((blk, _EMBED), lambda i: (i, 0))
             for _ in embsums],
        out_specs=pl.BlockSpec((blk, _EMBED), lambda i: (i, 0)),
        out_shape=jax.ShapeDtypeStruct((_BATCH, _EMBED), jnp.float32),
    )(cf_mat, w1t, b1.reshape(1, -1), w2t, b2.reshape(1, -1), *embsums)


def kernel(cf_0, cf_1, cf_2, cf_3, cf_4, cf_5, cf_6, cf_7,
           df_0, df_1, df_2, df_3, df_4, df_5, df_6, df_7, df_8, df_9,
           df_10, df_11, df_12, df_13, df_14, df_15, df_16, df_17, df_18,
           df_19, df_20, df_21, df_22, df_23, df_24, df_25,
           W1, b1, W2, b2, tables):
    cfs = [cf_0, cf_1, cf_2, cf_3, cf_4, cf_5, cf_6, cf_7]
    dfs = [df_0, df_1, df_2, df_3, df_4, df_5, df_6, df_7, df_8, df_9,
           df_10, df_11, df_12, df_13, df_14, df_15, df_16, df_17, df_18,
           df_19, df_20, df_21, df_22, df_23, df_24, df_25]
    cf_mat = jnp.stack(cfs, axis=1)
    didxf = (jnp.stack(dfs, axis=0)
             .reshape(_NUM_DF, _NW, _BPW)
             .transpose(1, 0, 2)
             .reshape(_NW * _NUM_DF * _BPW))
    tables_t = jnp.transpose(tables, (0, 2, 1))
    embsums = []
    for g in range(_NGRP):
        tables_pg = _tc_format_table(tables_t, g)
        embsums.append(_sc_embsum(didxf, tables_pg, g))
    return _tc_mlp(cf_mat, W1.T, b1, W2.T, b2, embsums)


# speedup vs baseline: 6.9259x; 1.0077x over previous
import functools

import jax
import jax.numpy as jnp
from jax import lax
from jax.experimental import pallas as pl
from jax.experimental.pallas import tpu as pltpu
from jax.experimental.pallas import tpu_sc as plsc

_NUM_CF = 8
_NUM_DF = 26
_VOCAB = 100000
_EMBED = 64
_BATCH = 16384

_INFO = plsc.get_sparse_core_info()
_NC = _INFO.num_cores
_NS = _INFO.num_subcores
_NW = _NC * _NS
_BPW = _BATCH // _NW
_IDXW = 128
_STG = 64
_GRP = 2
_NGRP = _NUM_DF // _GRP
_NSTG = (_GRP * _BPW) // _STG

_SPLIT = 49920
_PROWS = _VOCAB - _SPLIT
_ZPAD = 2048
_PROWSP = _PROWS + _ZPAD
_DUMMY = _PROWS
_CHUNK = 2048
_NFULL = _PROWS // _CHUNK
_TAIL = _PROWS - _NFULL * _CHUNK


def _xpose_body(ina_ref, inb_ref, out_hbm, ybuf0, ybuf1, sem0, sem1):
    i = pl.program_id(0)
    ybufs = (ybuf0, ybuf1)
    sems = (sem0, sem1)

    prev = [None, None]
    for k in range(_NFULL + 2):
        b = k % 2
        a = k * _CHUNK
        if prev[b] is not None:
            prev[b].wait()
        if k <= _NFULL:
            n = _CHUNK if k < _NFULL else _TAIL
            y = jnp.transpose(
                jnp.concatenate(
                    [ina_ref[0, :, a:a + n],
                     inb_ref[0, :, a:a + n],
                     ina_ref[0, :, _SPLIT + a:_SPLIT + a + n],
                     inb_ref[0, :, _SPLIT + a:_SPLIT + a + n]],
                    axis=0), (1, 0))
            ybufs[b][0:n, :] = y
            dst = out_hbm.at[i, pl.ds(a, n)]
        else:
            n = _ZPAD
            ybufs[b][0:n, :] = jnp.zeros((_ZPAD, 2 * _EMBED), jnp.float32)
            dst = out_hbm.at[i, pl.ds(_PROWS, n)]
            assert _ZPAD <= _CHUNK
        cp = pltpu.make_async_copy(ybufs[b].at[pl.ds(0, n)], dst, sems[b])
        cp.start()
        prev[b] = cp
    for b in range(2):
        if prev[b] is not None:
            prev[b].wait()


def _tc_format_table(tables_t, g):
    he = _EMBED // 2
    return pl.pallas_call(
        _xpose_body,
        grid=(_GRP,),
        in_specs=[
            pl.BlockSpec((1, he, _VOCAB),
                         lambda i, g=g: (g * _GRP + i, 0, 0)),
            pl.BlockSpec((1, he, _VOCAB),
                         lambda i, g=g: (g * _GRP + i, 1, 0)),
        ],
        out_specs=pl.BlockSpec(memory_space=pl.ANY),
        out_shape=jax.ShapeDtypeStruct((_GRP, _PROWSP, 2 * _EMBED),
                                       jnp.float32),
        scratch_shapes=[
            pltpu.VMEM((_CHUNK, 2 * _EMBED), jnp.float32),
            pltpu.VMEM((_CHUNK, 2 * _EMBED), jnp.float32),
            pltpu.SemaphoreType.DMA,
            pltpu.SemaphoreType.DMA,
        ],
        compiler_params=pltpu.CompilerParams(
            vmem_limit_bytes=60 * 1024 * 1024),
    )(tables_t, tables_t)


def _sc_embsum(didxf, tabs, feat_off):
    mesh = plsc.VectorSubcoreMesh(core_axis_name="c", subcore_axis_name="s")
    nops = len(tabs)
    nfeat = _GRP * nops
    stg_per_op = _GRP * (_BPW // _STG)

    @functools.partial(
        pl.kernel,
        out_type=jax.ShapeDtypeStruct((_BATCH, _EMBED), jnp.float32),
        mesh=mesh,
        scratch_types=[
            pltpu.VMEM((nfeat * _BPW,), jnp.int32),
            pltpu.VMEM((_STG,), jnp.int32),
            pltpu.VMEM((_STG,), jnp.int32),
            pltpu.VMEM((_STG,), jnp.int32),
            pltpu.VMEM((_STG,), jnp.int32),
            pltpu.VMEM((_STG, 2 * _EMBED), jnp.float32),
            pltpu.VMEM((_STG, 2 * _EMBED), jnp.float32),
            pltpu.VMEM((_STG, 2 * _EMBED), jnp.float32),
            pltpu.VMEM((_STG, 2 * _EMBED), jnp.float32),
            pltpu.VMEM((_BPW, _EMBED), jnp.float32),
            pltpu.SemaphoreType.DMA,
            pltpu.SemaphoreType.DMA,
        ],
    )
    def body(*refs):
        didx_hbm = refs[0]
        tab_hbms = refs[1:1 + nops]
        out_hbm = refs[1 + nops]
        (rawbig, lo0, lo1, hi0, hi1, rlo0, rlo1, rhi0, rhi1, acc,
         sem0, sem1) = refs[2 + nops:]
        wid = lax.axis_index("s") * _NC + lax.axis_index("c")
        lo_bufs = (lo0, lo1)
        hi_bufs = (hi0, hi1)
        rlo_bufs = (rlo0, rlo1)
        rhi_bufs = (rhi0, rhi1)
        sems = (sem0, sem1)
        per_feat = _BPW // _STG

        pltpu.sync_copy(
            didx_hbm.at[pl.ds(wid * (_NUM_DF * _BPW) + feat_off * _BPW,
                              nfeat * _BPW)],
            rawbig)

        zero16 = jnp.zeros((16,), jnp.float32)

        def stage_and_fire(h, slot):
            i = (h // per_feat) % _GRP
            lob, hib = lo_bufs[slot], hi_bufs[slot]
            for c in range(_STG // 16):
                v = rawbig[pl.ds(h * _STG + c * 16, 16)]
                is_lo = v < _SPLIT
                dm = _DUMMY + (v & (_ZPAD - 1))
                lob[pl.ds(c * 16, 16)] = jnp.where(is_lo, v, dm)
                hib[pl.ds(c * 16, 16)] = jnp.where(is_lo, dm,
                                                   v - _SPLIT)
            for t in range(nops):

                @pl.when(h // stg_per_op == t)
                def _(t=t):
                    pltpu.make_async_copy(
                        tab_hbms[t].at[i].at[lob], rlo_bufs[slot],
                        sems[slot]).start()
                    pltpu.make_async_copy(
                        tab_hbms[t].at[i].at[hib], rhi_bufs[slot],
                        sems[slot]).start()

        def wait_gathers(h, slot):
            i = (h // per_feat) % _GRP
            for t in range(nops):

                @pl.when(h // stg_per_op == t)
                def _(t=t):
                    pltpu.make_async_copy(
                        tab_hbms[t].at[i].at[lo_bufs[slot]],
                        rlo_bufs[slot], sems[slot]).wait()
                    pltpu.make_async_copy(
                        tab_hbms[t].at[i].at[hi_bufs[slot]],
                        rhi_bufs[slot], sems[slot]).wait()

        def accumulate(h, slot):
            rlo, rhi = rlo_bufs[slot], rhi_bufs[slot]
            sub = h % per_feat

            def accrow(r, _):
                arow = sub * _STG + r
                for c in range(_EMBED // 16):
                    plsc.addupdate(
                        acc.at[arow, pl.ds(c * 16, 16)],
                        rlo[r, pl.ds(c * 16, 16)]
                        + rhi[r, pl.ds(_EMBED + c * 16, 16)])
                return 0

            lax.fori_loop(0, _STG, accrow, 0, unroll=4)

        nstg = nfeat * per_feat
        stage_and_fire(0, 0)

        def zrow(r, _):
            for c in range(_EMBED // 16):
                acc[r, pl.ds(c * 16, 16)] = zero16
            return 0

        lax.fori_loop(0, _BPW, zrow, 0, unroll=4)

        def loop_body(h, _):
            for slot in range(2):

                @pl.when(h % 2 == slot)
                def _(slot=slot):
                    wait_gathers(h, slot)
                    @pl.when(h + 1 < nstg)
                    def _():
                        stage_and_fire(h + 1, 1 - slot)
                    accumulate(h, slot)

            return 0

        lax.fori_loop(0, nstg, loop_body, 0)
        pltpu.sync_copy(acc, out_hbm.at[pl.ds(wid * _BPW, _BPW)])

    return body(didxf, *tabs)


def _tc_mlp_body(cf_ref, w1_ref, b1_ref, w2_ref, b2_ref, *rest):
    emb_refs, out_ref = rest[:-1], rest[-1]
    x = cf_ref[...]
    x = jnp.where(jnp.isnan(x), 0.0, x)
    h = jnp.maximum(
        jnp.dot(x, w1_ref[...], preferred_element_type=jnp.float32)
        + b1_ref[...], 0.0)
    h = jnp.clip(h, -65000.0, 65000.0)
    o = jnp.maximum(
        jnp.dot(h, w2_ref[...], preferred_element_type=jnp.float32)
        + b2_ref[...], 0.0)
    for e in emb_refs:
        o = o + e[...]
    out_ref[...] = o


def _tc_mlp(cf_mat, w1t, b1, w2t, b2, embsums):
    blk = 2048
    grid = _BATCH // blk
    return pl.pallas_call(
        _tc_mlp_body,
        grid=(grid,),
        in_specs=[
            pl.BlockSpec((blk, _NUM_CF), lambda i: (i, 0)),
            pl.BlockSpec((_NUM_CF, 2 * _NUM_CF), lambda i: (0, 0)),
            pl.BlockSpec((1, 2 * _NUM_CF), lambda i: (0, 0)),
            pl.BlockSpec((2 * _NUM_CF, _EMBED), lambda i: (0, 0)),
            pl.BlockSpec((1, _EMBED), lambda i: (0, 0)),
        ] + [pl.BlockSpec((blk, _EMBED), lambda i: (i, 0))
             for _ in embsums],
        out_specs=pl.BlockSpec((blk, _EMBED), lambda i: (i, 0)),
        out_shape=jax.ShapeDtypeStruct((_BATCH, _EMBED), jnp.float32),
    )(cf_mat, w1t, b1.reshape(1, -1), w2t, b2.reshape(1, -1), *embsums)


def kernel(cf_0, cf_1, cf_2, cf_3, cf_4, cf_5, cf_6, cf_7,
           df_0, df_1, df_2, df_3, df_4, df_5, df_6, df_7, df_8, df_9,
           df_10, df_11, df_12, df_13, df_14, df_15, df_16, df_17, df_18,
           df_19, df_20, df_21, df_22, df_23, df_24, df_25,
           W1, b1, W2, b2, tables):
    cfs = [cf_0, cf_1, cf_2, cf_3, cf_4, cf_5, cf_6, cf_7]
    dfs = [df_0, df_1, df_2, df_3, df_4, df_5, df_6, df_7, df_8, df_9,
           df_10, df_11, df_12, df_13, df_14, df_15, df_16, df_17, df_18,
           df_19, df_20, df_21, df_22, df_23, df_24, df_25]
    cf_mat = jnp.stack(cfs, axis=1)
    didxf = (jnp.stack(dfs, axis=0)
             .reshape(_NUM_DF, _NW, _BPW)
             .transpose(1, 0, 2)
             .reshape(_NW * _NUM_DF * _BPW))
    tables_t = jnp.transpose(tables, (0, 2, 1))
    tc_outs = [_tc_format_table(tables_t, g) for g in range(_NGRP)]
    sc_groups = [(0, 2), (2, 2), (4, 2), (6, 2), (8, 2), (10, 3)]
    embsums = []
    for t0, n in sc_groups:
        embsums.append(
            _sc_embsum(didxf, tc_outs[t0:t0 + n], t0 * _GRP))
    return _tc_mlp(cf_mat, W1.T, b1, W2.T, b2, embsums)


# speedup vs baseline: 7.2479x; 1.0465x over previous
import functools

import jax
import jax.numpy as jnp
from jax import lax
from jax.experimental import pallas as pl
from jax.experimental.pallas import tpu as pltpu
from jax.experimental.pallas import tpu_sc as plsc

_NUM_CF = 8
_NUM_DF = 26
_VOCAB = 100000
_EMBED = 64
_BATCH = 16384

_INFO = plsc.get_sparse_core_info()
_NC = _INFO.num_cores
_NS = _INFO.num_subcores
_NW = _NC * _NS
_BPW = _BATCH // _NW
_IDXW = 128
_STG = 64
_GROUPS = ((0, 7), (7, 7), (14, 6), (20, 4), (24, 2))

_SPLIT = 49920
_PROWS = _VOCAB - _SPLIT
_ZPAD = 2048
_PROWSP = _PROWS + _ZPAD
_DUMMY = _PROWS
_CHUNK = 2048
_NFULL = _PROWS // _CHUNK
_TAIL = _PROWS - _NFULL * _CHUNK


def _xpose_body(in_ref, out_hbm, ybuf0, ybuf1, sem0, sem1):
    i = pl.program_id(0)
    ybufs = (ybuf0, ybuf1)
    sems = (sem0, sem1)
    x = in_ref

    prev = [None, None]
    for k in range(_NFULL + 2):
        b = k % 2
        a = k * _CHUNK
        if prev[b] is not None:
            prev[b].wait()
        if k <= _NFULL:
            n = _CHUNK if k < _NFULL else _TAIL
            y = jnp.transpose(
                jnp.concatenate(
                    [x[0, :, a:a + n],
                     x[0, :, _SPLIT + a:_SPLIT + a + n]],
                    axis=0), (1, 0))
            ybufs[b][0:n, :] = y
            dst = out_hbm.at[i, pl.ds(a, n)]
        else:
            n = _ZPAD
            ybufs[b][0:n, :] = jnp.zeros((_ZPAD, 2 * _EMBED), jnp.float32)
            dst = out_hbm.at[i, pl.ds(_PROWS, n)]
            assert _ZPAD <= _CHUNK
        cp = pltpu.make_async_copy(ybufs[b].at[pl.ds(0, n)], dst, sems[b])
        cp.start()
        prev[b] = cp
    for b in range(2):
        if prev[b] is not None:
            prev[b].wait()


def _tc_format_table(tables_t, t0, n):
    return pl.pallas_call(
        _xpose_body,
        grid=(n,),
        in_specs=[pl.BlockSpec((1, _EMBED, _VOCAB),
                               lambda i, t0=t0: (t0 + i, 0, 0))],
        out_specs=pl.BlockSpec(memory_space=pl.ANY),
        out_shape=jax.ShapeDtypeStruct((n, _PROWSP, 2 * _EMBED),
                                       jnp.float32),
        scratch_shapes=[
            pltpu.VMEM((_CHUNK, 2 * _EMBED), jnp.float32),
            pltpu.VMEM((_CHUNK, 2 * _EMBED), jnp.float32),
            pltpu.SemaphoreType.DMA,
            pltpu.SemaphoreType.DMA,
        ],
        compiler_params=pltpu.CompilerParams(
            vmem_limit_bytes=60 * 1024 * 1024),
    )(tables_t)


def _sc_embsum(didxf, tab, feat_off):
    mesh = plsc.VectorSubcoreMesh(core_axis_name="c", subcore_axis_name="s")
    nfeat = tab.shape[0]

    @functools.partial(
        pl.kernel,
        out_type=jax.ShapeDtypeStruct((_BATCH, _EMBED), jnp.float32),
        mesh=mesh,
        scratch_types=[
            pltpu.VMEM((nfeat * _BPW,), jnp.int32),
            pltpu.VMEM((_STG,), jnp.int32),
            pltpu.VMEM((_STG,), jnp.int32),
            pltpu.VMEM((_STG,), jnp.int32),
            pltpu.VMEM((_STG,), jnp.int32),
            pltpu.VMEM((_STG, 2 * _EMBED), jnp.float32),
            pltpu.VMEM((_STG, 2 * _EMBED), jnp.float32),
            pltpu.VMEM((_STG, 2 * _EMBED), jnp.float32),
            pltpu.VMEM((_STG, 2 * _EMBED), jnp.float32),
            pltpu.VMEM((_BPW, _EMBED), jnp.float32),
            pltpu.SemaphoreType.DMA,
            pltpu.SemaphoreType.DMA,
        ],
    )
    def body(didx_hbm, tab_hbm, out_hbm,
             rawbig, lo0, lo1, hi0, hi1, rlo0, rlo1, rhi0, rhi1, acc,
             sem0, sem1):
        wid = lax.axis_index("s") * _NC + lax.axis_index("c")
        lo_bufs = (lo0, lo1)
        hi_bufs = (hi0, hi1)
        rlo_bufs = (rlo0, rlo1)
        rhi_bufs = (rhi0, rhi1)
        sems = (sem0, sem1)
        per_feat = _BPW // _STG

        pltpu.sync_copy(
            didx_hbm.at[pl.ds(wid * (_NUM_DF * _BPW) + feat_off * _BPW,
                              nfeat * _BPW)],
            rawbig)

        zero16 = jnp.zeros((16,), jnp.float32)

        def stage_and_fire(h, slot):
            i = h // per_feat
            lob, hib = lo_bufs[slot], hi_bufs[slot]
            for c in range(_STG // 16):
                v = rawbig[pl.ds(h * _STG + c * 16, 16)]
                is_lo = v < _SPLIT
                dm = _DUMMY + (v & (_ZPAD - 1))
                lob[pl.ds(c * 16, 16)] = jnp.where(is_lo, v, dm)
                hib[pl.ds(c * 16, 16)] = jnp.where(is_lo, dm,
                                                   v - _SPLIT)
            pltpu.make_async_copy(
                tab_hbm.at[i].at[lob], rlo_bufs[slot], sems[slot]).start()
            pltpu.make_async_copy(
                tab_hbm.at[i].at[hib], rhi_bufs[slot], sems[slot]).start()

        def wait_gathers(h, slot):
            i = h // per_feat
            pltpu.make_async_copy(
                tab_hbm.at[i].at[lo_bufs[slot]], rlo_bufs[slot],
                sems[slot]).wait()
            pltpu.make_async_copy(
                tab_hbm.at[i].at[hi_bufs[slot]], rhi_bufs[slot],
                sems[slot]).wait()

        def accumulate(h, slot):
            rlo, rhi = rlo_bufs[slot], rhi_bufs[slot]
            sub = h % per_feat

            def accrow(r, _):
                arow = sub * _STG + r
                for c in range(_EMBED // 16):
                    plsc.addupdate(
                        acc.at[arow, pl.ds(c * 16, 16)],
                        rlo[r, pl.ds(c * 16, 16)]
                        + rhi[r, pl.ds(_EMBED + c * 16, 16)])
                return 0

            lax.fori_loop(0, _STG, accrow, 0, unroll=4)

        nstg = nfeat * per_feat
        stage_and_fire(0, 0)

        def zrow(r, _):
            for c in range(_EMBED // 16):
                acc[r, pl.ds(c * 16, 16)] = zero16
            return 0

        lax.fori_loop(0, _BPW, zrow, 0, unroll=4)

        def loop_body(h, _):
            for slot in range(2):

                @pl.when(h % 2 == slot)
                def _(slot=slot):
                    wait_gathers(h, slot)
                    @pl.when(h + 1 < nstg)
                    def _():
                        stage_and_fire(h + 1, 1 - slot)
                    accumulate(h, slot)

            return 0

        lax.fori_loop(0, nstg, loop_body, 0)
        pltpu.sync_copy(acc, out_hbm.at[pl.ds(wid * _BPW, _BPW)])

    return body(didxf, tab)


def _tc_mlp_body(cf_ref, w1_ref, b1_ref, w2_ref, b2_ref, *rest):
    emb_refs, out_ref = rest[:-1], rest[-1]
    x = cf_ref[...]
    x = jnp.where(jnp.isnan(x), 0.0, x)
    h = jnp.maximum(
        jnp.dot(x, w1_ref[...], preferred_element_type=jnp.float32)
        + b1_ref[...], 0.0)
    h = jnp.clip(h, -65000.0, 65000.0)
    o = jnp.maximum(
        jnp.dot(h, w2_ref[...], preferred_element_type=jnp.float32)
        + b2_ref[...], 0.0)
    for e in emb_refs:
        o = o + e[...]
    out_ref[...] = o


def _tc_mlp(cf_mat, w1t, b1, w2t, b2, embsums):
    blk = 2048
    grid = _BATCH // blk
    return pl.pallas_call(
        _tc_mlp_body,
        grid=(grid,),
        in_specs=[
            pl.BlockSpec((blk, _NUM_CF), lambda i: (i, 0)),
            pl.BlockSpec((_NUM_CF, 2 * _NUM_CF), lambda i: (0, 0)),
            pl.BlockSpec((1, 2 * _NUM_CF), lambda i: (0, 0)),
            pl.BlockSpec((2 * _NUM_CF, _EMBED), lambda i: (0, 0)),
            pl.BlockSpec((1, _EMBED), lambda i: (0, 0)),
        ] + [pl.BlockSpec((blk, _EMBED), lambda i: (i, 0))
             for _ in embsums],
        out_specs=pl.BlockSpec((blk, _EMBED), lambda i: (i, 0)),
        out_shape=jax.ShapeDtypeStruct((_BATCH, _EMBED), jnp.float32),
    )(cf_mat, w1t, b1.reshape(1, -1), w2t, b2.reshape(1, -1), *embsums)


def kernel(cf_0, cf_1, cf_2, cf_3, cf_4, cf_5, cf_6, cf_7,
           df_0, df_1, df_2, df_3, df_4, df_5, df_6, df_7, df_8, df_9,
           df_10, df_11, df_12, df_13, df_14, df_15, df_16, df_17, df_18,
           df_19, df_20, df_21, df_22, df_23, df_24, df_25,
           W1, b1, W2, b2, tables):
    cfs = [cf_0, cf_1, cf_2, cf_3, cf_4, cf_5, cf_6, cf_7]
    dfs = [df_0, df_1, df_2, df_3, df_4, df_5, df_6, df_7, df_8, df_9,
           df_10, df_11, df_12, df_13, df_14, df_15, df_16, df_17, df_18,
           df_19, df_20, df_21, df_22, df_23, df_24, df_25]
    cf_mat = jnp.stack(cfs, axis=1)
    didxf = (jnp.stack(dfs, axis=0)
             .reshape(_NUM_DF, _NW, _BPW)
             .transpose(1, 0, 2)
             .reshape(_NW * _NUM_DF * _BPW))
    tables_t = jnp.transpose(tables, (0, 2, 1))
    embsums = []
    for t0, n in _GROUPS:
        tab_g = _tc_format_table(tables_t, t0, n)
        embsums.append(_sc_embsum(didxf, tab_g, t0))
    return _tc_mlp(cf_mat, W1.T, b1, W2.T, b2, embsums)


# speedup vs baseline: 8.1838x; 1.1291x over previous
import functools

import jax
import jax.numpy as jnp
from jax import lax
from jax.experimental import pallas as pl
from jax.experimental.pallas import tpu as pltpu
from jax.experimental.pallas import tpu_sc as plsc

_NUM_CF = 8
_NUM_DF = 26
_VOCAB = 100000
_EMBED = 64
_BATCH = 16384

_INFO = plsc.get_sparse_core_info()
_NC = _INFO.num_cores
_NS = _INFO.num_subcores
_NW = _NC * _NS
_BPW = _BATCH // _NW
_IDXW = 128
_STG = 64
_GROUPS = ((0, 7), (7, 7), (14, 6), (20, 4), (24, 2))

_SPLIT = 49920
_PROWS = _VOCAB - _SPLIT
_ZPAD = 2048
_PROWSP = _PROWS + _ZPAD
_DUMMY = _PROWS
_CHUNK = 2048
_NFULL = _PROWS // _CHUNK
_TAIL = _PROWS - _NFULL * _CHUNK


def _xpose_body(in_ref, out_hbm, ybuf0, ybuf1, sem0, sem1):
    i = pl.program_id(0)
    ybufs = (ybuf0, ybuf1)
    sems = (sem0, sem1)
    x = in_ref

    prev = [None, None]
    for k in range(_NFULL + 2):
        b = k % 2
        a = k * _CHUNK
        if prev[b] is not None:
            prev[b].wait()
        if k <= _NFULL:
            n = _CHUNK if k < _NFULL else _TAIL
            y = jnp.transpose(
                jnp.concatenate(
                    [x[0, :, a:a + n],
                     x[0, :, _SPLIT + a:_SPLIT + a + n]],
                    axis=0), (1, 0))
            ybufs[b][0:n, :] = y
            dst = out_hbm.at[i, pl.ds(a, n)]
        else:
            n = _ZPAD
            ybufs[b][0:n, :] = jnp.zeros((_ZPAD, 2 * _EMBED), jnp.float32)
            dst = out_hbm.at[i, pl.ds(_PROWS, n)]
            assert _ZPAD <= _CHUNK
        cp = pltpu.make_async_copy(ybufs[b].at[pl.ds(0, n)], dst, sems[b])
        cp.start()
        prev[b] = cp
    for b in range(2):
        if prev[b] is not None:
            prev[b].wait()


def _tc_format_table(tables_t, t0, n):
    return pl.pallas_call(
        _xpose_body,
        grid=(n,),
        in_specs=[pl.BlockSpec((1, _EMBED, _VOCAB),
                               lambda i, t0=t0: (t0 + i, 0, 0))],
        out_specs=pl.BlockSpec(memory_space=pl.ANY),
        out_shape=jax.ShapeDtypeStruct((n, _PROWSP, 2 * _EMBED),
                                       jnp.float32),
        scratch_shapes=[
            pltpu.VMEM((_CHUNK, 2 * _EMBED), jnp.float32),
            pltpu.VMEM((_CHUNK, 2 * _EMBED), jnp.float32),
            pltpu.SemaphoreType.DMA,
            pltpu.SemaphoreType.DMA,
        ],
        compiler_params=pltpu.CompilerParams(
            vmem_limit_bytes=60 * 1024 * 1024),
    )(tables_t)


def _sc_embsum(didxf, tab, feat_off):
    mesh = plsc.VectorSubcoreMesh(core_axis_name="c", subcore_axis_name="s")
    nfeat = tab.shape[0]

    @functools.partial(
        pl.kernel,
        out_type=jax.ShapeDtypeStruct((_BATCH, _EMBED), jnp.float32),
        mesh=mesh,
        scratch_types=[
            pltpu.VMEM((nfeat * _BPW,), jnp.int32),
            pltpu.VMEM((_STG,), jnp.int32),
            pltpu.VMEM((_STG,), jnp.int32),
            pltpu.VMEM((_STG, 2 * _EMBED), jnp.float32),
            pltpu.VMEM((_STG, 2 * _EMBED), jnp.float32),
            pltpu.VMEM((_BPW, _EMBED), jnp.float32),
            pltpu.SemaphoreType.DMA,
            pltpu.SemaphoreType.DMA,
        ],
    )
    def body(didx_hbm, tab_hbm, out_hbm,
             rawbig, ip0, ip1, rw0, rw1, acc, sem0, sem1):
        wid = lax.axis_index("s") * _NC + lax.axis_index("c")
        ip_bufs = (ip0, ip1)
        rw_bufs = (rw0, rw1)
        sems = (sem0, sem1)
        per_feat = _BPW // _STG

        pltpu.sync_copy(
            didx_hbm.at[pl.ds(wid * (_NUM_DF * _BPW) + feat_off * _BPW,
                              nfeat * _BPW)],
            rawbig)

        zero16 = jnp.zeros((16,), jnp.float32)

        def stage_and_fire(h, slot):
            i = h // per_feat
            ipb = ip_bufs[slot]
            for c in range(_STG // 16):
                v = rawbig[pl.ds(h * _STG + c * 16, 16)]
                ipb[pl.ds(c * 16, 16)] = jnp.where(v < _SPLIT, v,
                                                   v - _SPLIT)
            pltpu.make_async_copy(
                tab_hbm.at[i].at[ipb], rw_bufs[slot], sems[slot]).start()

        def wait_gathers(h, slot):
            i = h // per_feat
            pltpu.make_async_copy(
                tab_hbm.at[i].at[ip_bufs[slot]], rw_bufs[slot],
                sems[slot]).wait()

        def accumulate(h, slot):
            rb = rw_bufs[slot]
            sub = h % per_feat

            def accblk(j, _):
                v16 = rawbig[pl.ds(h * _STG + j * 16, 16)]
                for rr in range(16):
                    off = jnp.where(v16[rr] < _SPLIT, 0, _EMBED)
                    r = j * 16 + rr
                    arow = sub * _STG + r
                    for c in range(_EMBED // 16):
                        plsc.addupdate(
                            acc.at[arow, pl.ds(c * 16, 16)],
                            rb[r, pl.ds(off + c * 16, 16)])
                return 0

            lax.fori_loop(0, _STG // 16, accblk, 0)

        nstg = nfeat * per_feat
        stage_and_fire(0, 0)

        def zrow(r, _):
            for c in range(_EMBED // 16):
                acc[r, pl.ds(c * 16, 16)] = zero16
            return 0

        lax.fori_loop(0, _BPW, zrow, 0, unroll=4)

        def loop_body(h, _):
            for slot in range(2):

                @pl.when(h % 2 == slot)
                def _(slot=slot):
                    wait_gathers(h, slot)
                    @pl.when(h + 1 < nstg)
                    def _():
                        stage_and_fire(h + 1, 1 - slot)
                    accumulate(h, slot)

            return 0

        lax.fori_loop(0, nstg, loop_body, 0)
        pltpu.sync_copy(acc, out_hbm.at[pl.ds(wid * _BPW, _BPW)])

    return body(didxf, tab)


def _tc_mlp_body(cf_ref, w1_ref, b1_ref, w2_ref, b2_ref, *rest):
    emb_refs, out_ref = rest[:-1], rest[-1]
    x = cf_ref[...]
    x = jnp.where(jnp.isnan(x), 0.0, x)
    h = jnp.maximum(
        jnp.dot(x, w1_ref[...], preferred_element_type=jnp.float32)
        + b1_ref[...], 0.0)
    h = jnp.clip(h, -65000.0, 65000.0)
    o = jnp.maximum(
        jnp.dot(h, w2_ref[...], preferred_element_type=jnp.float32)
        + b2_ref[...], 0.0)
    for e in emb_refs:
        o = o + e[...]
    out_ref[...] = o


def _tc_mlp(cf_mat, w1t, b1, w2t, b2, embsums):
    blk = 2048
    grid = _BATCH // blk
    return pl.pallas_call(
        _tc_mlp_body,
        grid=(grid,),
        in_specs=[
            pl.BlockSpec((blk, _NUM_CF), lambda i: (i, 0)),
            pl.BlockSpec((_NUM_CF, 2 * _NUM_CF), lambda i: (0, 0)),
            pl.BlockSpec((1, 2 * _NUM_CF), lambda i: (0, 0)),
            pl.BlockSpec((2 * _NUM_CF, _EMBED), lambda i: (0, 0)),
            pl.BlockSpec((1, _EMBED), lambda i: (0, 0)),
        ] + [pl.BlockSpec((blk, _EMBED), lambda i: (i, 0))
             for _ in embsums],
        out_specs=pl.BlockSpec((blk, _EMBED), lambda i: (i, 0)),
        out_shape=jax.ShapeDtypeStruct((_BATCH, _EMBED), jnp.float32),
    )(cf_mat, w1t, b1.reshape(1, -1), w2t, b2.reshape(1, -1), *embsums)


def kernel(cf_0, cf_1, cf_2, cf_3, cf_4, cf_5, cf_6, cf_7,
           df_0, df_1, df_2, df_3, df_4, df_5, df_6, df_7, df_8, df_9,
           df_10, df_11, df_12, df_13, df_14, df_15, df_16, df_17, df_18,
           df_19, df_20, df_21, df_22, df_23, df_24, df_25,
           W1, b1, W2, b2, tables):
    cfs = [cf_0, cf_1, cf_2, cf_3, cf_4, cf_5, cf_6, cf_7]
    dfs = [df_0, df_1, df_2, df_3, df_4, df_5, df_6, df_7, df_8, df_9,
           df_10, df_11, df_12, df_13, df_14, df_15, df_16, df_17, df_18,
           df_19, df_20, df_21, df_22, df_23, df_24, df_25]
    cf_mat = jnp.stack(cfs, axis=1)
    didxf = (jnp.stack(dfs, axis=0)
             .reshape(_NUM_DF, _NW, _BPW)
             .transpose(1, 0, 2)
             .reshape(_NW * _NUM_DF * _BPW))
    tables_t = jnp.transpose(tables, (0, 2, 1))
    embsums = []
    for t0, n in _GROUPS:
        tab_g = _tc_format_table(tables_t, t0, n)
        embsums.append(_sc_embsum(didxf, tab_g, t0))
    return _tc_mlp(cf_mat, W1.T, b1, W2.T, b2, embsums)


# speedup vs baseline: 8.3614x; 1.0217x over previous
import functools

import jax
import jax.numpy as jnp
from jax import lax
from jax.experimental import pallas as pl
from jax.experimental.pallas import tpu as pltpu
from jax.experimental.pallas import tpu_sc as plsc

_NUM_CF = 8
_NUM_DF = 26
_VOCAB = 100000
_EMBED = 64
_BATCH = 16384

_INFO = plsc.get_sparse_core_info()
_NC = _INFO.num_cores
_NS = _INFO.num_subcores
_NW = _NC * _NS
_BPW = _BATCH // _NW
_IDXW = 128
_STG = 64
_GROUPS = ((0, 3), (3, 6), (9, 6), (15, 5), (20, 4), (24, 2))

_SPLIT = 49920
_PROWS = _VOCAB - _SPLIT
_PROWSP = _PROWS
_CHUNK = 2048
_NFULL = _PROWS // _CHUNK
_TAIL = _PROWS - _NFULL * _CHUNK


def _xpose_body(in_ref, out_hbm, ybuf0, ybuf1, sem0, sem1):
    i = pl.program_id(0)
    ybufs = (ybuf0, ybuf1)
    sems = (sem0, sem1)
    x = in_ref

    prev = [None, None]
    for k in range(_NFULL + 1):
        b = k % 2
        a = k * _CHUNK
        n = _CHUNK if k < _NFULL else _TAIL
        if prev[b] is not None:
            prev[b].wait()
        y = jnp.transpose(
            jnp.concatenate(
                [x[0, :, a:a + n],
                 x[0, :, _SPLIT + a:_SPLIT + a + n]],
                axis=0), (1, 0))
        ybufs[b][0:n, :] = y
        cp = pltpu.make_async_copy(
            ybufs[b].at[pl.ds(0, n)], out_hbm.at[i, pl.ds(a, n)], sems[b])
        cp.start()
        prev[b] = cp
    for b in range(2):
        if prev[b] is not None:
            prev[b].wait()


def _tc_format_table(tables_t, t0, n):
    return pl.pallas_call(
        _xpose_body,
        grid=(n,),
        in_specs=[pl.BlockSpec((1, _EMBED, _VOCAB),
                               lambda i, t0=t0: (t0 + i, 0, 0))],
        out_specs=pl.BlockSpec(memory_space=pl.ANY),
        out_shape=jax.ShapeDtypeStruct((n, _PROWSP, 2 * _EMBED),
                                       jnp.float32),
        scratch_shapes=[
            pltpu.VMEM((_CHUNK, 2 * _EMBED), jnp.float32),
            pltpu.VMEM((_CHUNK, 2 * _EMBED), jnp.float32),
            pltpu.SemaphoreType.DMA,
            pltpu.SemaphoreType.DMA,
        ],
        compiler_params=pltpu.CompilerParams(
            vmem_limit_bytes=60 * 1024 * 1024),
    )(tables_t)


def _sc_embsum(didxf, tab, feat_off):
    mesh = plsc.VectorSubcoreMesh(core_axis_name="c", subcore_axis_name="s")
    nfeat = tab.shape[0]

    @functools.partial(
        pl.kernel,
        out_type=jax.ShapeDtypeStruct((_BATCH, _EMBED), jnp.float32),
        mesh=mesh,
        scratch_types=[
            pltpu.VMEM((nfeat * _BPW,), jnp.int32),
            pltpu.VMEM((_STG,), jnp.int32),
            pltpu.VMEM((_STG,), jnp.int32),
            pltpu.VMEM((_STG, 2 * _EMBED), jnp.float32),
            pltpu.VMEM((_STG, 2 * _EMBED), jnp.float32),
            pltpu.VMEM((_BPW, _EMBED), jnp.float32),
            pltpu.SemaphoreType.DMA,
            pltpu.SemaphoreType.DMA,
        ],
    )
    def body(didx_hbm, tab_hbm, out_hbm,
             rawbig, ip0, ip1, rw0, rw1, acc, sem0, sem1):
        wid = lax.axis_index("s") * _NC + lax.axis_index("c")
        ip_bufs = (ip0, ip1)
        rw_bufs = (rw0, rw1)
        sems = (sem0, sem1)
        per_feat = _BPW // _STG

        pltpu.sync_copy(
            didx_hbm.at[pl.ds(wid * (_NUM_DF * _BPW) + feat_off * _BPW,
                              nfeat * _BPW)],
            rawbig)

        zero16 = jnp.zeros((16,), jnp.float32)

        def stage_and_fire(h, slot):
            i = h // per_feat
            ipb = ip_bufs[slot]
            for c in range(_STG // 16):
                v = rawbig[pl.ds(h * _STG + c * 16, 16)]
                ipb[pl.ds(c * 16, 16)] = jnp.where(v < _SPLIT, v,
                                                   v - _SPLIT)
            pltpu.make_async_copy(
                tab_hbm.at[i].at[ipb], rw_bufs[slot], sems[slot]).start()

        def wait_gathers(h, slot):
            i = h // per_feat
            pltpu.make_async_copy(
                tab_hbm.at[i].at[ip_bufs[slot]], rw_bufs[slot],
                sems[slot]).wait()

        def accumulate(h, slot):
            rb = rw_bufs[slot]
            sub = h % per_feat

            def accblk(j, _):
                v16 = rawbig[pl.ds(h * _STG + j * 16, 16)]
                for rr in range(16):
                    off = jnp.where(v16[rr] < _SPLIT, 0, _EMBED)
                    r = j * 16 + rr
                    arow = sub * _STG + r
                    for c in range(_EMBED // 16):
                        plsc.addupdate(
                            acc.at[arow, pl.ds(c * 16, 16)],
                            rb[r, pl.ds(off + c * 16, 16)])
                return 0

            lax.fori_loop(0, _STG // 16, accblk, 0)

        nstg = nfeat * per_feat
        stage_and_fire(0, 0)

        def zrow(r, _):
            for c in range(_EMBED // 16):
                acc[r, pl.ds(c * 16, 16)] = zero16
            return 0

        lax.fori_loop(0, _BPW, zrow, 0, unroll=4)

        def loop_body(h, _):
            for slot in range(2):

                @pl.when(h % 2 == slot)
                def _(slot=slot):
                    wait_gathers(h, slot)
                    @pl.when(h + 1 < nstg)
                    def _():
                        stage_and_fire(h + 1, 1 - slot)
                    accumulate(h, slot)

            return 0

        lax.fori_loop(0, nstg, loop_body, 0)
        pltpu.sync_copy(acc, out_hbm.at[pl.ds(wid * _BPW, _BPW)])

    return body(didxf, tab)


def _tc_mlp_body(cf_ref, w1_ref, b1_ref, w2_ref, b2_ref, *rest):
    emb_refs, out_ref = rest[:-1], rest[-1]
    x = cf_ref[...]
    x = jnp.where(jnp.isnan(x), 0.0, x)
    h = jnp.maximum(
        jnp.dot(x, w1_ref[...], preferred_element_type=jnp.float32)
        + b1_ref[...], 0.0)
    h = jnp.clip(h, -65000.0, 65000.0)
    o = jnp.maximum(
        jnp.dot(h, w2_ref[...], preferred_element_type=jnp.float32)
        + b2_ref[...], 0.0)
    for e in emb_refs:
        o = o + e[...]
    out_ref[...] = o


def _tc_mlp(cf_mat, w1t, b1, w2t, b2, embsums):
    blk = 2048
    grid = _BATCH // blk
    return pl.pallas_call(
        _tc_mlp_body,
        grid=(grid,),
        in_specs=[
            pl.BlockSpec((blk, _NUM_CF), lambda i: (i, 0)),
            pl.BlockSpec((_NUM_CF, 2 * _NUM_CF), lambda i: (0, 0)),
            pl.BlockSpec((1, 2 * _NUM_CF), lambda i: (0, 0)),
            pl.BlockSpec((2 * _NUM_CF, _EMBED), lambda i: (0, 0)),
            pl.BlockSpec((1, _EMBED), lambda i: (0, 0)),
        ] + [pl.BlockSpec((blk, _EMBED), lambda i: (i, 0))
             for _ in embsums],
        out_specs=pl.BlockSpec((blk, _EMBED), lambda i: (i, 0)),
        out_shape=jax.ShapeDtypeStruct((_BATCH, _EMBED), jnp.float32),
    )(cf_mat, w1t, b1.reshape(1, -1), w2t, b2.reshape(1, -1), *embsums)


def kernel(cf_0, cf_1, cf_2, cf_3, cf_4, cf_5, cf_6, cf_7,
           df_0, df_1, df_2, df_3, df_4, df_5, df_6, df_7, df_8, df_9,
           df_10, df_11, df_12, df_13, df_14, df_15, df_16, df_17, df_18,
           df_19, df_20, df_21, df_22, df_23, df_24, df_25,
           W1, b1, W2, b2, tables):
    cfs = [cf_0, cf_1, cf_2, cf_3, cf_4, cf_5, cf_6, cf_7]
    dfs = [df_0, df_1, df_2, df_3, df_4, df_5, df_6, df_7, df_8, df_9,
           df_10, df_11, df_12, df_13, df_14, df_15, df_16, df_17, df_18,
           df_19, df_20, df_21, df_22, df_23, df_24, df_25]
    cf_mat = jnp.stack(cfs, axis=1)
    didxf = (jnp.stack(dfs, axis=0)
             .reshape(_NUM_DF, _NW, _BPW)
             .transpose(1, 0, 2)
             .reshape(_NW * _NUM_DF * _BPW))
    tables_t = jnp.transpose(tables, (0, 2, 1))
    embsums = []
    for t0, n in _GROUPS:
        tab_g = _tc_format_table(tables_t, t0, n)
        embsums.append(_sc_embsum(didxf, tab_g, t0))
    return _tc_mlp(cf_mat, W1.T, b1, W2.T, b2, embsums)
